# baseline (speedup 1.0000x reference)
import jax
import jax.numpy as jnp
from jax import lax
from jax.experimental import pallas as pl

_ANCH_W = (10.0, 16.0, 33.0)
_ANCH_H = (13.0, 30.0, 23.0)
_GS = 52
_G = _GS * _GS
_NA = 3
_NF = 85
_STRIDE = 8.0


def _body(x_ref, o_ref):
    a = pl.program_id(1)
    v = x_ref[0, 0].astype(jnp.float32)

    aw = jnp.where(a == 0, _ANCH_W[0], jnp.where(a == 1, _ANCH_W[1], _ANCH_W[2]))
    ah = jnp.where(a == 0, _ANCH_H[0], jnp.where(a == 1, _ANCH_H[1], _ANCH_H[2]))

    g = lax.broadcasted_iota(jnp.int32, (2, _G), 1)
    r = lax.broadcasted_iota(jnp.int32, (2, _G), 0)
    grid_off = jnp.where(r == 0, g % _GS, g // _GS).astype(jnp.float32)

    xy = (jax.nn.sigmoid(v[0:2, :]) + grid_off) * _STRIDE
    wh = jnp.exp(v[2:4, :]) * jnp.where(
        lax.broadcasted_iota(jnp.int32, (2, _G), 0) == 0, aw, ah)
    rest = jax.nn.sigmoid(v[4:, :])

    full = jnp.concatenate(
        [xy, wh, rest, jnp.zeros((128 - _NF, _G), jnp.float32)], axis=0)
    o_ref[0] = full.T.astype(jnp.bfloat16)


def kernel(inputs):
    b = inputs.shape[0]
    x = inputs.astype(jnp.bfloat16).reshape(b, _NA, _NF, _G)
    out = pl.pallas_call(
        _body,
        grid=(b, _NA),
        in_specs=[pl.BlockSpec((1, 1, _NF, _G), lambda i, j: (i, j, 0, 0))],
        out_specs=pl.BlockSpec((1, _G, 128), lambda i, j: (i, j, 0)),
        out_shape=jax.ShapeDtypeStruct((b, _NA * _G, 128), jnp.bfloat16),
    )(x)
    return (out[:, :, :_NF].astype(jnp.float32), 0, 0)


# speedup vs baseline: 1.0792x; 1.0792x over previous
import jax
import jax.numpy as jnp
from jax import lax
from jax.experimental import pallas as pl

_ANCH_W = (10.0, 16.0, 33.0)
_ANCH_H = (13.0, 30.0, 23.0)
_GS = 52
_G = _GS * _GS
_NA = 3
_NF = 85
_STRIDE = 8.0


def _body(x_ref, o_ref):
    a = pl.program_id(1)
    v = x_ref[0, 0].astype(jnp.float32)

    aw = jnp.where(a == 0, _ANCH_W[0], jnp.where(a == 1, _ANCH_W[1], _ANCH_W[2]))
    ah = jnp.where(a == 0, _ANCH_H[0], jnp.where(a == 1, _ANCH_H[1], _ANCH_H[2]))

    g = lax.broadcasted_iota(jnp.int32, (2, _G), 1)
    r = lax.broadcasted_iota(jnp.int32, (2, _G), 0)
    grid_off = jnp.where(r == 0, g % _GS, g // _GS).astype(jnp.float32)

    xy = (jax.nn.sigmoid(v[0:2, :]) + grid_off) * _STRIDE
    wh = jnp.exp(v[2:4, :]) * jnp.where(
        lax.broadcasted_iota(jnp.int32, (2, _G), 0) == 0, aw, ah)
    rest = jax.nn.sigmoid(v[4:, :])

    full = jnp.concatenate(
        [xy, wh, rest, jnp.zeros((128 - _NF, _G), jnp.float32)], axis=0)
    o_ref[0] = full.T


def kernel(inputs):
    b = inputs.shape[0]
    x = inputs.astype(jnp.bfloat16).reshape(b, _NA, _NF, _G)
    out = pl.pallas_call(
        _body,
        grid=(b, _NA),
        in_specs=[pl.BlockSpec((1, 1, _NF, _G), lambda i, j: (i, j, 0, 0))],
        out_specs=pl.BlockSpec((1, _G, 128), lambda i, j: (i, j, 0)),
        out_shape=jax.ShapeDtypeStruct((b, _NA * _G, 128), jnp.float32),
    )(x)
    return (out[:, :, :_NF], 0, 0)


# speedup vs baseline: 1.0795x; 1.0002x over previous
import jax
import jax.numpy as jnp
from jax import lax
from jax.experimental import pallas as pl

_ANCH_W = (10.0, 16.0, 33.0)
_ANCH_H = (13.0, 30.0, 23.0)
_GS = 52
_G = _GS * _GS
_NA = 3
_NF = 85
_STRIDE = 8.0


def _body(x_ref, o_ref):
    a = pl.program_id(1)
    v = x_ref[0, 0].astype(jnp.float32)

    aw = jnp.where(a == 0, _ANCH_W[0], jnp.where(a == 1, _ANCH_W[1], _ANCH_W[2]))
    ah = jnp.where(a == 0, _ANCH_H[0], jnp.where(a == 1, _ANCH_H[1], _ANCH_H[2]))

    g = lax.broadcasted_iota(jnp.int32, (2, _G), 1)
    r = lax.broadcasted_iota(jnp.int32, (2, _G), 0)
    grid_off = jnp.where(r == 0, g % _GS, g // _GS).astype(jnp.float32)

    xy = (jax.nn.sigmoid(v[0:2, :]) + grid_off) * _STRIDE
    wh = jnp.exp(v[2:4, :]) * jnp.where(
        lax.broadcasted_iota(jnp.int32, (2, _G), 0) == 0, aw, ah)
    rest = jax.nn.sigmoid(v[4:, :])

    full = jnp.concatenate([xy, wh, rest], axis=0)
    o_ref[0] = full.T


def kernel(inputs):
    b = inputs.shape[0]
    x = inputs.astype(jnp.bfloat16).reshape(b, _NA, _NF, _G)
    out = pl.pallas_call(
        _body,
        grid=(b, _NA),
        in_specs=[pl.BlockSpec((1, 1, _NF, _G), lambda i, j: (i, j, 0, 0))],
        out_specs=pl.BlockSpec((1, _G, _NF), lambda i, j: (i, j, 0)),
        out_shape=jax.ShapeDtypeStruct((b, _NA * _G, _NF), jnp.float32),
    )(x)
    return (out, 0, 0)
